# baseline (speedup 1.0000x reference)
import functools

import jax
import jax.numpy as jnp
from jax import lax
from jax.experimental import pallas as pl
from jax.experimental.pallas import tpu as pltpu
from jax.experimental.pallas import tpu_sc as plsc

VOCAB = 100000
EMBED = 64
BATCH = 4096
HIST = 50
B = BATCH * HIST

_info = plsc.get_sparse_core_info()
NC = _info.num_cores
NS = _info.num_subcores
NW = NC * NS
BPW = B // NW
CH = 800
NCHUNK = BPW // CH


@functools.partial(
    pl.kernel,
    mesh=plsc.VectorSubcoreMesh(core_axis_name="c", subcore_axis_name="s"),
    out_type=jax.ShapeDtypeStruct((BATCH, HIST, EMBED), jnp.float32),
    scratch_types=[
        pltpu.VMEM((BPW,), jnp.int32),
        pltpu.VMEM((2, CH, EMBED), jnp.float32),
        pltpu.SemaphoreType.DMA,
        pltpu.SemaphoreType.DMA,
    ],
    compiler_params=pltpu.CompilerParams(use_tc_tiling_on_sc=False),
)
def _gather_kernel(x_hbm, table_hbm, out_hbm, idx_v, rows_v, gsem, ssem):
    wid = lax.axis_index("s") * NC + lax.axis_index("c")
    base = wid * BPW
    pltpu.sync_copy(x_hbm.at[pl.ds(base, BPW)], idx_v)

    def gather(c):
        return pltpu.async_copy(
            table_hbm.at[idx_v.at[pl.ds(c * CH, CH)]], rows_v.at[c % 2], gsem)

    RPC = CH // HIST

    def store(c):
        b0 = wid * (BPW // HIST) + c * RPC
        cps = []
        for r in range(RPC):
            cps.append(pltpu.async_copy(
                rows_v.at[c % 2].at[pl.ds(r * HIST, HIST)],
                out_hbm.at[b0 + r], ssem))
        return cps

    stores = [None] * NCHUNK
    g = gather(0)
    for c in range(NCHUNK):
        g.wait()
        stores[c] = store(c)
        if c + 1 < NCHUNK:
            if c >= 1:
                for cp in stores[c - 1]:
                    cp.wait()
            g = gather(c + 1)
    for cp in stores[NCHUNK - 1]:
        cp.wait()
    if NCHUNK >= 2:
        for cp in stores[NCHUNK - 2]:
            cp.wait()


def kernel(x, table):
    flat = x.reshape(B)
    return _gather_kernel(flat, table)
